# Initial kernel scaffold; baseline (speedup 1.0000x reference)
import jax
import jax.numpy as jnp
from jax.experimental import pallas as pl


def kernel(x, edge_index, sage0_Wl, sage0_bl, sage0_Wr, lin0_W, lin0_b, sage1_Wl, sage1_bl, sage1_Wr, lin1_W, lin1_b):
    raise NotImplementedError("write your pallas kernel here")


# speedup vs baseline: 8.7630x; 8.7630x over previous
import functools

import jax
import jax.numpy as jnp
from jax import lax
from jax.experimental import pallas as pl
from jax.experimental.pallas import tpu as pltpu
from jax.experimental.pallas import tpu_sc as plsc

_N = 10000
_E = 160000
_NCORES = 2
_NTILES = 16
_NPAD = _N
_RPT = _NPAD // _NTILES


def _make_segsum(width, nch, ch, with_counts, split_edges, two_tables):
  out_types = [jax.ShapeDtypeStruct((_NCORES, _NPAD, width), jnp.float32)]
  scratch = [
      pltpu.VMEM_SHARED((_NPAD, width), jnp.float32),
      pltpu.VMEM((nch, ch), jnp.int32),
      pltpu.VMEM((nch, ch), jnp.int32),
      pltpu.VMEM((ch, width), jnp.float32),
      pltpu.VMEM((ch, width), jnp.float32),
      pltpu.SemaphoreType.DMA,
      pltpu.SemaphoreType.DMA,
  ]
  if with_counts:
    out_types.append(jax.ShapeDtypeStruct((_NPAD, 16), jnp.float32))
    scratch += [
        pltpu.VMEM_SHARED((_NPAD, 16), jnp.float32),
        pltpu.VMEM((ch, 16), jnp.float32),
    ]
  mesh = plsc.VectorSubcoreMesh(core_axis_name="c", subcore_axis_name="s")
  n_tables = 2 if two_tables else 1

  def body(*refs):
    it = iter(refs)
    tables = [next(it) for _ in range(n_tables)]
    src_hbm = next(it)
    dst_hbm = next(it)
    zeros_hbm = next(it)
    if with_counts:
      zcnt_hbm = next(it)
      ones_hbm = next(it)
    msg_hbm = next(it)
    if with_counts:
      cnt_hbm = next(it)
    acc_sh = next(it)
    src_v = next(it)
    dst_v = next(it)
    rows = (next(it), next(it))
    sems = (next(it), next(it))
    if with_counts:
      cnt_sh = next(it)
      ones_v = next(it)

    c = lax.axis_index("c")
    s = lax.axis_index("s")

    pltpu.sync_copy(zeros_hbm.at[pl.ds(s * _RPT, _RPT)],
                    acc_sh.at[pl.ds(s * _RPT, _RPT)])
    if with_counts:
      @pl.when(c == 0)
      def _():
        pltpu.sync_copy(zcnt_hbm.at[pl.ds(s * _RPT, _RPT)],
                        cnt_sh.at[pl.ds(s * _RPT, _RPT)])
      pltpu.sync_copy(ones_hbm, ones_v)
    if split_edges:
      pltpu.sync_copy(src_hbm.at[c, s], src_v)
      pltpu.sync_copy(dst_hbm.at[c, s], dst_v)
    else:
      pltpu.sync_copy(src_hbm.at[s], src_v)
      pltpu.sync_copy(dst_hbm.at[s], dst_v)
    plsc.subcore_barrier()

    def start_gather(j, b):
      if two_tables:
        @pl.when(c == 0)
        def _():
          pltpu.async_copy(tables[0].at[src_v.at[j]], rows[b], sems[b])

        @pl.when(c == 1)
        def _():
          pltpu.async_copy(tables[1].at[src_v.at[j]], rows[b], sems[b])
      else:
        pltpu.async_copy(tables[0].at[src_v.at[j]], rows[b], sems[b])

    def consume(j, b):
      pltpu.make_async_copy(tables[0].at[src_v.at[j]], rows[b],
                            sems[b]).wait()
      nxt = j + 2

      @pl.when(nxt < nch)
      def _():
        start_gather(nxt, b)

      pltpu.sync_copy(rows[b], acc_sh.at[dst_v.at[j]], add=True)
      if with_counts:
        @pl.when(c == 0)
        def _():
          pltpu.sync_copy(ones_v, cnt_sh.at[dst_v.at[j]], add=True)

    start_gather(0, 0)
    if nch > 1:
      start_gather(1, 1)
    main = nch if nch % 2 == 0 else nch - 1

    @pl.loop(0, main, step=2)
    def _(k):
      for b in range(2):
        consume(k + b, b)

    if nch % 2 == 1:
      consume(nch - 1, 0)

    plsc.subcore_barrier()
    pltpu.sync_copy(acc_sh.at[pl.ds(s * _RPT, _RPT)],
                    msg_hbm.at[c, pl.ds(s * _RPT, _RPT)])
    if with_counts:
      @pl.when(c == 0)
      def _():
        pltpu.sync_copy(cnt_sh.at[pl.ds(s * _RPT, _RPT)],
                        cnt_hbm.at[pl.ds(s * _RPT, _RPT)])

  return pl.kernel(body, out_type=tuple(out_types), mesh=mesh,
                   scratch_types=scratch,
                   compiler_params=pltpu.CompilerParams(
                       use_tc_tiling_on_sc=False))


_segsum0 = _make_segsum(width=128, nch=250, ch=40, with_counts=True,
                        split_edges=False, two_tables=True)
_segsum1 = _make_segsum(width=48, nch=125, ch=40, with_counts=False,
                        split_edges=True, two_tables=False)

_RB = 1000


def _dense0_body(msg_ref, cnt_ref, x_ref, wa_ref, wr_ref, b0_ref,
                 w2a_ref, w2b_ref, y1_ref, y2_ref):
  acc = jnp.dot(msg_ref[0], wa_ref[:128, :], preferred_element_type=jnp.float32)
  acc = acc + jnp.dot(msg_ref[1], wa_ref[128:, :],
                      preferred_element_type=jnp.float32)
  cnt = jnp.max(cnt_ref[...], axis=1, keepdims=True)
  inv = 1.0 / jnp.maximum(cnt, 1.0)
  h = acc * inv + jnp.dot(x_ref[...], wr_ref[...],
                          preferred_element_type=jnp.float32) + b0_ref[...]
  x1 = jnp.maximum(h, 0.0)
  y1_ref[...] = jnp.dot(x1, w2a_ref[...], preferred_element_type=jnp.float32)
  y2_ref[...] = jnp.dot(x1, w2b_ref[...], preferred_element_type=jnp.float32)


_dense0 = pl.pallas_call(
    _dense0_body,
    grid=(_N // _RB,),
    in_specs=[
        pl.BlockSpec((_NCORES, _RB, 128), lambda i: (0, i, 0)),
        pl.BlockSpec((_RB, 16), lambda i: (i, 0)),
        pl.BlockSpec((_RB, 256), lambda i: (i, 0)),
        pl.BlockSpec((256, 256), lambda i: (0, 0)),
        pl.BlockSpec((256, 256), lambda i: (0, 0)),
        pl.BlockSpec((1, 256), lambda i: (0, 0)),
        pl.BlockSpec((256, 48), lambda i: (0, 0)),
        pl.BlockSpec((256, 48), lambda i: (0, 0)),
    ],
    out_specs=[
        pl.BlockSpec((_RB, 48), lambda i: (i, 0)),
        pl.BlockSpec((_RB, 48), lambda i: (i, 0)),
    ],
    out_shape=[
        jax.ShapeDtypeStruct((_N, 48), jnp.float32),
        jax.ShapeDtypeStruct((_N, 48), jnp.float32),
    ],
)


def _dense1_body(msg_ref, cnt_ref, y2_ref, c1_ref, out_ref):
  ssum = msg_ref[0] + msg_ref[1]
  cnt = jnp.max(cnt_ref[...], axis=1, keepdims=True)
  inv = 1.0 / jnp.maximum(cnt, 1.0)
  res = ssum * inv + y2_ref[...] + c1_ref[...]
  out_ref[...] = res[:, :40]


_dense1 = pl.pallas_call(
    _dense1_body,
    grid=(_N // _RB,),
    in_specs=[
        pl.BlockSpec((_NCORES, _RB, 48), lambda i: (0, i, 0)),
        pl.BlockSpec((_RB, 16), lambda i: (i, 0)),
        pl.BlockSpec((_RB, 48), lambda i: (i, 0)),
        pl.BlockSpec((1, 48), lambda i: (0, 0)),
    ],
    out_specs=pl.BlockSpec((_RB, 40), lambda i: (i, 0)),
    out_shape=jax.ShapeDtypeStruct((_N, 40), jnp.float32),
)


def kernel(x, edge_index, sage0_Wl, sage0_bl, sage0_Wr, lin0_W, lin0_b,
           sage1_Wl, sage1_bl, sage1_Wr, lin1_W, lin1_b):
  src = edge_index[0].astype(jnp.int32)
  dst = edge_index[1].astype(jnp.int32)

  x_lo = x[:, :128]
  x_hi = x[:, 128:]
  srcA = src.reshape(_NTILES, 250, 40)
  dstA = dst.reshape(_NTILES, 250, 40)
  zeros128 = jnp.zeros((_NPAD, 128), jnp.float32)
  zeros16 = jnp.zeros((_NPAD, 16), jnp.float32)
  ones = jnp.ones((40, 16), jnp.float32)
  msg0, cnt = _segsum0(x_lo, x_hi, srcA, dstA, zeros128, zeros16, ones)

  wa = sage0_Wl.T
  wr = sage0_Wr.T
  b0 = (sage0_bl + lin0_b).reshape(1, 256)
  w2a = jnp.pad(sage1_Wl.T, ((0, 0), (0, 8)))
  w2b = jnp.pad(sage1_Wr.T, ((0, 0), (0, 8)))
  y1, y2 = _dense0(msg0, cnt, x, wa, wr, b0, w2a, w2b)

  srcC = src.reshape(_NCORES, _NTILES, 125, 40)
  dstC = dst.reshape(_NCORES, _NTILES, 125, 40)
  zeros48 = jnp.zeros((_NPAD, 48), jnp.float32)
  (msg1,) = _segsum1(y1, srcC, dstC, zeros48)

  c1 = (sage1_bl + lin1_W @ lin0_b + lin1_b)
  c1p = jnp.pad(c1, (0, 8)).reshape(1, 48)
  return _dense1(msg1, cnt, y2, c1p)
